# baseline (speedup 1.0000x reference)
import functools
import jax
import jax.numpy as jnp
from jax import lax
from jax.experimental import pallas as pl
from jax.experimental.pallas import tpu as pltpu
from jax.experimental.pallas import tpu_sc as plsc

VOCAB = 100000
EMBED_DIM = 64
BATCH = 16384

_NC = 2
_NS = 16
_NW = _NC * _NS
_B_PER_W = BATCH // _NW
_CHUNK = 128
_N_CHUNKS = _B_PER_W // _CHUNK
_L = 16

_TC_FULL = VOCAB // 128
_TC_REM = VOCAB - _TC_FULL * 128
_A_ITERS = -(-_TC_FULL // _NW)

_mesh = plsc.VectorSubcoreMesh(core_axis_name="c", subcore_axis_name="s")
_params = pltpu.CompilerParams(needs_layout_passes=False)


@functools.partial(
    pl.kernel,
    out_type=jax.ShapeDtypeStruct((VOCAB // 2, 2 * EMBED_DIM), jnp.float32),
    mesh=_mesh,
    scratch_types=[
        pltpu.VMEM((2, EMBED_DIM, 128), jnp.float32),
        pltpu.VMEM((2, EMBED_DIM, 128), jnp.float32),
        pltpu.SemaphoreType.DMA,
    ],
    compiler_params=_params,
)
def _transpose_table(wt_hbm, tail_hbm, out_hbm, in_v, tr_v, isem):
    wid = lax.axis_index("s") * _NC + lax.axis_index("c")
    iota = lax.iota(jnp.int32, _L)
    i0s = [(g % 4) * 16 + iota for g in range(8)]

    def fire(k, buf):
        tc = k * _NW + wid

        @pl.when(tc < _TC_FULL)
        def _():
            pltpu.async_copy(
                wt_hbm.at[:, pl.ds(tc * 128, 128)], in_v.at[buf], isem
            )

    def shuffle(buf):
        src = in_v.at[buf]
        dst = tr_v.at[buf]

        def body(j, _):
            for u in range(2):
                r = j * 2 + u
                r2 = r * 2
                for g in range(8):
                    x = plsc.load_gather(
                        src, [i0s[g], lax.broadcast(r2 + g // 4, (_L,))]
                    )
                    dst[r, pl.ds(g * 16, _L)] = x
            return 0

        lax.fori_loop(0, EMBED_DIM // 2, body, 0)

    fire(0, 0)
    for k in range(_A_ITERS):
        buf = k % 2
        fire(k + 1, (k + 1) % 2)
        tc = k * _NW + wid

        @pl.when(tc < _TC_FULL)
        def _():
            pltpu.make_async_copy(
                wt_hbm.at[:, pl.ds(0, 128)], in_v.at[buf], isem
            ).wait()
            shuffle(buf)
            pltpu.sync_copy(
                tr_v.at[buf], out_hbm.at[pl.ds(tc * EMBED_DIM, EMBED_DIM)]
            )

    @pl.when(wid == _NW - 1)
    def _():
        pltpu.sync_copy(tail_hbm, in_v.at[0, pl.ds(0, _TC_REM // 2)])
        pltpu.sync_copy(
            in_v.at[0, pl.ds(0, _TC_REM // 2)],
            out_hbm.at[pl.ds(_TC_FULL * EMBED_DIM, _TC_REM // 2)],
        )


@functools.partial(
    pl.kernel,
    out_type=jax.ShapeDtypeStruct((EMBED_DIM, BATCH), jnp.float32),
    mesh=_mesh,
    scratch_types=[
        pltpu.VMEM((_B_PER_W,), jnp.int32),
        pltpu.VMEM((_B_PER_W,), jnp.int32),
        pltpu.VMEM((_B_PER_W,), jnp.int32),
        pltpu.VMEM((2, _CHUNK, 128), jnp.float32),
        pltpu.VMEM((EMBED_DIM, _CHUNK), jnp.float32),
        pltpu.SemaphoreType.DMA,
    ],
    compiler_params=_params,
)
def _gather_rows(idx_hbm, table_hbm, out_hbm, idx_v, v_v, s_v, blocks_v,
                 ot_v, gsem):
    wid = lax.axis_index("s") * _NC + lax.axis_index("c")
    base = wid * _B_PER_W

    pltpu.sync_copy(idx_hbm.at[wid], idx_v)

    for k in range(_B_PER_W // _L):
        ivec = idx_v[pl.ds(k * _L, _L)]
        v_v[pl.ds(k * _L, _L)] = lax.shift_right_logical(ivec, 1)
        s_v[pl.ds(k * _L, _L)] = lax.shift_left(lax.bitwise_and(ivec, 1), 6)

    iota = lax.iota(jnp.int32, _L)
    i0s = [lg * 16 + iota for lg in range(_CHUNK // _L)]

    def fire(c, buf):
        return pltpu.async_copy(
            table_hbm.at[v_v.at[pl.ds(c * _CHUNK, _CHUNK)]],
            blocks_v.at[buf],
            gsem,
        )

    def extract(c, buf):
        src = blocks_v.at[buf]
        for lg in range(_CHUNK // _L):
            i0 = i0s[lg]
            s64 = s_v[pl.ds(c * _CHUNK + lg * 16, _L)]

            def body(j, _):
                for u in range(4):
                    d = j * 4 + u
                    x = plsc.load_gather(src, [i0, s64 + d])
                    plsc.store_scatter(
                        ot_v, [lax.broadcast(d, (_L,)), i0], x
                    )
                return 0

            lax.fori_loop(0, EMBED_DIM // 4, body, 0)

    cp = fire(0, 0)
    for c in range(_N_CHUNKS):
        nxt = fire(c + 1, (c + 1) % 2) if c + 1 < _N_CHUNKS else None
        cp.wait()
        extract(c, c % 2)
        pltpu.sync_copy(
            ot_v, out_hbm.at[:, pl.ds(base + c * _CHUNK, _CHUNK)]
        )
        cp = nxt


def kernel(inputs, W):
    idx = inputs.reshape(_NW, _B_PER_W)
    tail = W[_TC_FULL * 128:].reshape(_TC_REM // 2, 2 * EMBED_DIM)
    table = _transpose_table(W.T, tail)
    return _gather_rows(idx, table).T


# speedup vs baseline: 1.5753x; 1.5753x over previous
import functools
import jax
import jax.numpy as jnp
from jax import lax
from jax.experimental import pallas as pl
from jax.experimental.pallas import tpu as pltpu
from jax.experimental.pallas import tpu_sc as plsc

VOCAB = 100000
EMBED_DIM = 64
BATCH = 16384

_NC = 2
_NS = 16
_NW = _NC * _NS
_B_PER_W = BATCH // _NW
_CHUNK = 128
_N_CHUNKS = _B_PER_W // _CHUNK
_L = 16

_TC_FULL = VOCAB // 128
_TC_REM = VOCAB - _TC_FULL * 128
_A_ITERS = -(-_TC_FULL // _NW)

_mesh = plsc.VectorSubcoreMesh(core_axis_name="c", subcore_axis_name="s")
_params = pltpu.CompilerParams(needs_layout_passes=False)


@functools.partial(
    pl.kernel,
    out_type=jax.ShapeDtypeStruct((VOCAB // 2, 2 * EMBED_DIM), jnp.float32),
    mesh=_mesh,
    scratch_types=[
        pltpu.VMEM((2, EMBED_DIM, 128), jnp.float32),
        pltpu.VMEM((2, EMBED_DIM, 128), jnp.float32),
        pltpu.SemaphoreType.DMA,
    ],
    compiler_params=_params,
)
def _transpose_table(wt_hbm, tail_hbm, out_hbm, in_v, tr_v, isem):
    wid = lax.axis_index("s") * _NC + lax.axis_index("c")
    iota = lax.iota(jnp.int32, _L)
    i0s = [(g % 4) * 16 + iota for g in range(8)]

    def fire(k, buf):
        tc = k * _NW + wid

        @pl.when(tc < _TC_FULL)
        def _():
            pltpu.async_copy(
                wt_hbm.at[:, pl.ds(tc * 128, 128)], in_v.at[buf], isem
            )

    def shuffle(buf):
        src = in_v.at[buf]
        dst = tr_v.at[buf]

        @plsc.parallel_loop(0, EMBED_DIM, 1, unroll=8)
        def _(r):
            r2 = r * 2
            for g in range(8):
                x = plsc.load_gather(
                    src, [i0s[g], lax.broadcast(r2 + g // 4, (_L,))]
                )
                dst[r, pl.ds(g * 16, _L)] = x

    fire(0, 0)
    for k in range(_A_ITERS):
        buf = k % 2
        fire(k + 1, (k + 1) % 2)
        tc = k * _NW + wid

        @pl.when(tc < _TC_FULL)
        def _():
            pltpu.make_async_copy(
                wt_hbm.at[:, pl.ds(0, 128)], in_v.at[buf], isem
            ).wait()
            shuffle(buf)
            pltpu.sync_copy(
                tr_v.at[buf], out_hbm.at[pl.ds(tc * EMBED_DIM, EMBED_DIM)]
            )

    @pl.when(wid == _NW - 1)
    def _():
        pltpu.sync_copy(tail_hbm, in_v.at[0, pl.ds(0, _TC_REM // 2)])
        pltpu.sync_copy(
            in_v.at[0, pl.ds(0, _TC_REM // 2)],
            out_hbm.at[pl.ds(_TC_FULL * EMBED_DIM, _TC_REM // 2)],
        )


@functools.partial(
    pl.kernel,
    out_type=jax.ShapeDtypeStruct((EMBED_DIM, BATCH), jnp.float32),
    mesh=_mesh,
    scratch_types=[
        pltpu.VMEM((_B_PER_W,), jnp.int32),
        pltpu.VMEM((_B_PER_W,), jnp.int32),
        pltpu.VMEM((_B_PER_W,), jnp.int32),
        pltpu.VMEM((2, _CHUNK, 128), jnp.float32),
        pltpu.VMEM((EMBED_DIM, _CHUNK), jnp.float32),
        pltpu.SemaphoreType.DMA,
    ],
    compiler_params=_params,
)
def _gather_rows(idx_hbm, table_hbm, out_hbm, idx_v, v_v, s_v, blocks_v,
                 ot_v, gsem):
    wid = lax.axis_index("s") * _NC + lax.axis_index("c")
    base = wid * _B_PER_W

    pltpu.sync_copy(idx_hbm.at[wid], idx_v)

    for k in range(_B_PER_W // _L):
        ivec = idx_v[pl.ds(k * _L, _L)]
        v_v[pl.ds(k * _L, _L)] = lax.shift_right_logical(ivec, 1)
        s_v[pl.ds(k * _L, _L)] = lax.shift_left(lax.bitwise_and(ivec, 1), 6)

    iota = lax.iota(jnp.int32, _L)
    i0s = [lg * 16 + iota for lg in range(_CHUNK // _L)]

    def fire(c, buf):
        return pltpu.async_copy(
            table_hbm.at[v_v.at[pl.ds(c * _CHUNK, _CHUNK)]],
            blocks_v.at[buf],
            gsem,
        )

    def extract(c, buf):
        src = blocks_v.at[buf]
        for lg in range(_CHUNK // _L):
            i0 = i0s[lg]
            s64 = s_v[pl.ds(c * _CHUNK + lg * 16, _L)]

            @plsc.parallel_loop(0, EMBED_DIM, 1, unroll=8)
            def _(d):
                x = plsc.load_gather(src, [i0, s64 + d])
                plsc.store_scatter(ot_v, [lax.broadcast(d, (_L,)), i0], x)

    cp = fire(0, 0)
    for c in range(_N_CHUNKS):
        nxt = fire(c + 1, (c + 1) % 2) if c + 1 < _N_CHUNKS else None
        cp.wait()
        extract(c, c % 2)
        pltpu.sync_copy(
            ot_v, out_hbm.at[:, pl.ds(base + c * _CHUNK, _CHUNK)]
        )
        cp = nxt


def kernel(inputs, W):
    idx = inputs.reshape(_NW, _B_PER_W)
    tail = W[_TC_FULL * 128:].reshape(_TC_REM // 2, 2 * EMBED_DIM)
    table = _transpose_table(W.T, tail)
    return _gather_rows(idx, table).T


# speedup vs baseline: 2.2660x; 1.4385x over previous
import functools
import jax
import jax.numpy as jnp
from jax import lax
from jax.experimental import pallas as pl
from jax.experimental.pallas import tpu as pltpu
from jax.experimental.pallas import tpu_sc as plsc

VOCAB = 100000
EMBED_DIM = 64
BATCH = 16384

_NC = 2
_NS = 16
_NW = _NC * _NS
_B_PER_W = BATCH // _NW
_CHUNK = 128
_N_CHUNKS = _B_PER_W // _CHUNK
_L = 16

_TC_FULL = VOCAB // 128
_TC_REM = VOCAB - _TC_FULL * 128
_A_ITERS = -(-_TC_FULL // _NW)

_mesh = plsc.VectorSubcoreMesh(core_axis_name="c", subcore_axis_name="s")
_params = pltpu.CompilerParams(needs_layout_passes=False)


@functools.partial(
    pl.kernel,
    out_type=jax.ShapeDtypeStruct((VOCAB // 2, 2 * EMBED_DIM), jnp.float32),
    mesh=_mesh,
    scratch_types=[
        pltpu.VMEM((2, EMBED_DIM, 128), jnp.float32),
        pltpu.VMEM((2, EMBED_DIM, 128), jnp.float32),
        pltpu.SemaphoreType.DMA,
    ],
    compiler_params=_params,
)
def _transpose_table(wt_hbm, tail_hbm, out_hbm, in_v, tr_v, isem):
    wid = lax.axis_index("s") * _NC + lax.axis_index("c")
    iota = lax.iota(jnp.int32, _L)
    i0s = [(g % 4) * 16 + iota for g in range(8)]

    def fire(k, buf):
        tc = k * _NW + wid

        @pl.when(tc < _TC_FULL)
        def _():
            pltpu.async_copy(
                wt_hbm.at[:, pl.ds(tc * 128, 128)], in_v.at[buf], isem
            )

    def shuffle(buf):
        src = in_v.at[buf]
        dst = tr_v.at[buf]

        @plsc.parallel_loop(0, EMBED_DIM, 1, unroll=8)
        def _(r):
            r2 = r * 2
            for g in range(8):
                x = plsc.load_gather(
                    src, [i0s[g], lax.broadcast(r2 + g // 4, (_L,))]
                )
                dst[r, pl.ds(g * 16, _L)] = x

    fire(0, 0)
    for k in range(_A_ITERS):
        buf = k % 2
        fire(k + 1, (k + 1) % 2)
        tc = k * _NW + wid

        @pl.when(tc < _TC_FULL)
        def _():
            pltpu.make_async_copy(
                wt_hbm.at[:, pl.ds(0, 128)], in_v.at[buf], isem
            ).wait()
            shuffle(buf)
            pltpu.sync_copy(
                tr_v.at[buf], out_hbm.at[pl.ds(tc * EMBED_DIM, EMBED_DIM)]
            )

    @pl.when(wid == _NW - 1)
    def _():
        pltpu.sync_copy(tail_hbm, in_v.at[0, pl.ds(0, _TC_REM // 2)])
        pltpu.sync_copy(
            in_v.at[0, pl.ds(0, _TC_REM // 2)],
            out_hbm.at[pl.ds(_TC_FULL * EMBED_DIM, _TC_REM // 2)],
        )


@functools.partial(
    pl.kernel,
    out_type=jax.ShapeDtypeStruct((EMBED_DIM, BATCH), jnp.float32),
    mesh=_mesh,
    scratch_types=[
        pltpu.VMEM((_B_PER_W,), jnp.int32),
        pltpu.VMEM((_B_PER_W,), jnp.int32),
        pltpu.VMEM((_B_PER_W,), jnp.int32),
        pltpu.VMEM((2, _CHUNK, 128), jnp.float32),
        pltpu.VMEM((EMBED_DIM, _CHUNK), jnp.float32),
        pltpu.SemaphoreType.DMA,
    ],
    compiler_params=_params,
)
def _gather_rows(idx_hbm, table_hbm, out_hbm, idx_v, v_v, s_v, blocks_v,
                 ot_v, gsem):
    wid = lax.axis_index("s") * _NC + lax.axis_index("c")
    base = wid * _B_PER_W

    pltpu.sync_copy(idx_hbm.at[wid], idx_v)

    for k in range(_B_PER_W // _L):
        ivec = idx_v[pl.ds(k * _L, _L)]
        v_v[pl.ds(k * _L, _L)] = lax.shift_right_logical(ivec, 1)
        s_v[pl.ds(k * _L, _L)] = lax.shift_left(lax.bitwise_and(ivec, 1), 6)

    iota = lax.iota(jnp.int32, _L)
    i0s = [lg * 16 + iota for lg in range(_CHUNK // _L)]

    def fire(c, buf):
        return pltpu.async_copy(
            table_hbm.at[v_v.at[pl.ds(c * _CHUNK, _CHUNK)]],
            blocks_v.at[buf],
            gsem,
        )

    def extract(c, buf):
        src = blocks_v.at[buf]
        for lg in range(_CHUNK // _L):
            i0 = i0s[lg]
            s64 = s_v[pl.ds(c * _CHUNK + lg * 16, _L)]

            @plsc.parallel_loop(0, EMBED_DIM, 1, unroll=8)
            def _(d):
                x = plsc.load_gather(src, [i0, s64 + d])
                plsc.store_scatter(ot_v, [lax.broadcast(d, (_L,)), i0], x)

    cp = fire(0, 0)
    for c in range(_N_CHUNKS):
        nxt = fire(c + 1, (c + 1) % 2) if c + 1 < _N_CHUNKS else None
        cp.wait()
        extract(c, c % 2)
        pltpu.sync_copy(
            ot_v, out_hbm.at[:, pl.ds(base + c * _CHUNK, _CHUNK)]
        )
        cp = nxt


def kernel(inputs, W):
    idx = inputs.reshape(_NW, _B_PER_W)
    table = W.reshape(VOCAB // 2, 2 * EMBED_DIM)
    return _gather_rows(idx, table).T


# speedup vs baseline: 2.3304x; 1.0284x over previous
import functools
import jax
import jax.numpy as jnp
from jax import lax
from jax.experimental import pallas as pl
from jax.experimental.pallas import tpu as pltpu
from jax.experimental.pallas import tpu_sc as plsc

VOCAB = 100000
EMBED_DIM = 64
BATCH = 16384

_NC = 2
_NS = 16
_NW = _NC * _NS
_B_PER_W = BATCH // _NW
_CHUNK = 128
_N_CHUNKS = _B_PER_W // _CHUNK
_L = 16

_NGRP = VOCAB // 8
_G_PER_W = 392
_GBLK = 28
_A_ITERS = _G_PER_W // _GBLK
_G_TAIL0 = _GBLK * 12 + _G_PER_W * (_NW - 1)
_G_TAIL = _NGRP - _G_TAIL0

_mesh = plsc.VectorSubcoreMesh(core_axis_name="c", subcore_axis_name="s")
_params = pltpu.CompilerParams(needs_layout_passes=False)


@functools.partial(
    pl.kernel,
    out_type=jax.ShapeDtypeStruct((VOCAB // 32, 16, 128), jnp.float32),
    mesh=_mesh,
    scratch_types=[
        pltpu.VMEM((2, _GBLK, 8, EMBED_DIM), jnp.float32),
        pltpu.VMEM((2, _GBLK // 4, 16, 128), jnp.float32),
        pltpu.SemaphoreType.DMA,
    ],
    compiler_params=_params,
)
def _depad_table(wp_hbm, out_hbm, a_v, b_v, isem):
    wid = lax.axis_index("s") * _NC + lax.axis_index("c")
    g0 = wid * _G_PER_W
    gend = lax.min(g0 + _G_PER_W, jnp.int32(_G_TAIL0))

    def fire(k, buf, blk, gbase):
        g = gbase + k * _GBLK

        @pl.when(g + blk <= gend)
        def _():
            pltpu.async_copy(
                wp_hbm.at[pl.ds(g, blk)], a_v.at[buf, pl.ds(0, blk)], isem
            )

    def squeeze(buf, blk):
        src = a_v.at[buf]
        dst = b_v.at[buf]

        @plsc.parallel_loop(0, blk * 8, 1, unroll=8)
        def _(t):
            h = lax.bitwise_and(t, 1)
            p = lax.shift_right_logical(t, 1)
            v = lax.shift_right_logical(p, 4)
            s = lax.bitwise_and(p, 15)
            for c in range(EMBED_DIM // _L):
                x = src[lax.shift_right_logical(t, 3),
                        lax.bitwise_and(t, 7),
                        pl.ds(c * _L, _L)]
                dst[v, s, pl.ds(h * EMBED_DIM + c * _L, _L)] = x

    fire(0, 0, _GBLK, g0)
    for k in range(_A_ITERS):
        buf = k % 2
        fire(k + 1, (k + 1) % 2, _GBLK, g0)
        g = g0 + k * _GBLK

        @pl.when(g + _GBLK <= gend)
        def _():
            pltpu.make_async_copy(
                wp_hbm.at[pl.ds(0, _GBLK)], a_v.at[buf, pl.ds(0, _GBLK)],
                isem,
            ).wait()
            squeeze(buf, _GBLK)
            pltpu.sync_copy(
                b_v.at[buf],
                out_hbm.at[pl.ds(lax.shift_right_logical(g, 2), _GBLK // 4)],
            )

    @pl.when(wid == _NW - 1)
    def _():
        pltpu.sync_copy(
            wp_hbm.at[pl.ds(_G_TAIL0, _G_TAIL)],
            a_v.at[0, pl.ds(0, _G_TAIL)],
        )
        squeeze(0, _G_TAIL)
        pltpu.sync_copy(
            b_v.at[0, pl.ds(0, _G_TAIL // 4)],
            out_hbm.at[pl.ds(_G_TAIL0 // 4, _G_TAIL // 4)],
        )


@functools.partial(
    pl.kernel,
    out_type=jax.ShapeDtypeStruct((EMBED_DIM, BATCH), jnp.float32),
    mesh=_mesh,
    scratch_types=[
        pltpu.VMEM((_B_PER_W,), jnp.int32),
        pltpu.VMEM((_B_PER_W,), jnp.int32),
        pltpu.VMEM((_B_PER_W,), jnp.int32),
        pltpu.VMEM((2, _CHUNK, 128), jnp.float32),
        pltpu.VMEM((EMBED_DIM, _CHUNK), jnp.float32),
        pltpu.SemaphoreType.DMA,
    ],
    compiler_params=_params,
)
def _gather_rows(idx_hbm, table_hbm, out_hbm, idx_v, v_v, s_v, blocks_v,
                 ot_v, gsem):
    wid = lax.axis_index("s") * _NC + lax.axis_index("c")
    base = wid * _B_PER_W

    pltpu.sync_copy(idx_hbm.at[wid], idx_v)

    for k in range(_B_PER_W // _L):
        ivec = idx_v[pl.ds(k * _L, _L)]
        v_v[pl.ds(k * _L, _L)] = lax.shift_right_logical(ivec, 1)
        s_v[pl.ds(k * _L, _L)] = lax.shift_left(lax.bitwise_and(ivec, 1), 6)

    iota = lax.iota(jnp.int32, _L)
    i0s = [lg * 16 + iota for lg in range(_CHUNK // _L)]

    def fire(c, buf):
        return pltpu.async_copy(
            table_hbm.at[v_v.at[pl.ds(c * _CHUNK, _CHUNK)]],
            blocks_v.at[buf],
            gsem,
        )

    def extract(c, buf):
        src = blocks_v.at[buf]
        for lg in range(_CHUNK // _L):
            i0 = i0s[lg]
            s64 = s_v[pl.ds(c * _CHUNK + lg * 16, _L)]

            @plsc.parallel_loop(0, EMBED_DIM, 1, unroll=8)
            def _(d):
                x = plsc.load_gather(src, [i0, s64 + d])
                plsc.store_scatter(ot_v, [lax.broadcast(d, (_L,)), i0], x)

    cp = fire(0, 0)
    for c in range(_N_CHUNKS):
        nxt = fire(c + 1, (c + 1) % 2) if c + 1 < _N_CHUNKS else None
        cp.wait()
        extract(c, c % 2)
        pltpu.sync_copy(
            ot_v, out_hbm.at[:, pl.ds(base + c * _CHUNK, _CHUNK)]
        )
        cp = nxt


def kernel(inputs, W):
    idx = inputs.reshape(_NW, _B_PER_W)
    wp = W.reshape(_NGRP, 8, EMBED_DIM)
    table = _depad_table(wp).reshape(VOCAB // 2, 2 * EMBED_DIM)
    return _gather_rows(idx, table).T


# speedup vs baseline: 2.3475x; 1.0073x over previous
import functools
import jax
import jax.numpy as jnp
from jax import lax
from jax.experimental import pallas as pl
from jax.experimental.pallas import tpu as pltpu
from jax.experimental.pallas import tpu_sc as plsc

VOCAB = 100000
EMBED_DIM = 64
BATCH = 16384

_NC = 2
_NS = 16
_NW = _NC * _NS
_B_PER_W = BATCH // _NW
_CHUNK = 128
_N_CHUNKS = _B_PER_W // _CHUNK
_L = 16

_NGRP = VOCAB // 8
_G_PER_W = 392
_GBLK = 28
_A_ITERS = _G_PER_W // _GBLK
_G_TAIL0 = _GBLK * 12 + _G_PER_W * (_NW - 1)
_G_TAIL = _NGRP - _G_TAIL0

_mesh = plsc.VectorSubcoreMesh(core_axis_name="c", subcore_axis_name="s")
_params = pltpu.CompilerParams(needs_layout_passes=False)


@functools.partial(
    pl.kernel,
    out_type=jax.ShapeDtypeStruct((VOCAB // 32, 16, 128), jnp.float32),
    mesh=_mesh,
    scratch_types=[
        pltpu.VMEM((2, _GBLK, 8, EMBED_DIM), jnp.float32),
        pltpu.VMEM((2, _GBLK // 4, 16, 128), jnp.float32),
        pltpu.SemaphoreType.DMA,
    ],
    compiler_params=_params,
)
def _depad_table(wp_hbm, out_hbm, a_v, b_v, isem):
    wid = lax.axis_index("s") * _NC + lax.axis_index("c")
    g0 = wid * _G_PER_W
    gend = lax.min(g0 + _G_PER_W, jnp.int32(_G_TAIL0))

    def fire(k, buf, blk, gbase):
        g = gbase + k * _GBLK

        @pl.when(g + blk <= gend)
        def _():
            pltpu.async_copy(
                wp_hbm.at[pl.ds(g, blk)], a_v.at[buf, pl.ds(0, blk)], isem
            )

    def squeeze(buf, blk):
        src = a_v.at[buf]
        dst = b_v.at[buf]

        @plsc.parallel_loop(0, blk * 8, 1, unroll=8)
        def _(t):
            h = lax.bitwise_and(t, 1)
            p = lax.shift_right_logical(t, 1)
            v = lax.shift_right_logical(p, 4)
            s = lax.bitwise_and(p, 15)
            for c in range(EMBED_DIM // _L):
                x = src[lax.shift_right_logical(t, 3),
                        lax.bitwise_and(t, 7),
                        pl.ds(c * _L, _L)]
                dst[v, s, pl.ds(h * EMBED_DIM + c * _L, _L)] = x

    fire(0, 0, _GBLK, g0)
    for k in range(_A_ITERS):
        buf = k % 2
        fire(k + 1, (k + 1) % 2, _GBLK, g0)
        g = g0 + k * _GBLK

        @pl.when(g + _GBLK <= gend)
        def _():
            pltpu.make_async_copy(
                wp_hbm.at[pl.ds(0, _GBLK)], a_v.at[buf, pl.ds(0, _GBLK)],
                isem,
            ).wait()
            squeeze(buf, _GBLK)
            pltpu.sync_copy(
                b_v.at[buf],
                out_hbm.at[pl.ds(lax.shift_right_logical(g, 2), _GBLK // 4)],
            )

    @pl.when(wid == _NW - 1)
    def _():
        pltpu.sync_copy(
            wp_hbm.at[pl.ds(_G_TAIL0, _G_TAIL)],
            a_v.at[0, pl.ds(0, _G_TAIL)],
        )
        squeeze(0, _G_TAIL)
        pltpu.sync_copy(
            b_v.at[0, pl.ds(0, _G_TAIL // 4)],
            out_hbm.at[pl.ds(_G_TAIL0 // 4, _G_TAIL // 4)],
        )


@functools.partial(
    pl.kernel,
    out_type=jax.ShapeDtypeStruct((EMBED_DIM, BATCH), jnp.float32),
    mesh=_mesh,
    scratch_types=[
        pltpu.VMEM((_B_PER_W,), jnp.int32),
        pltpu.VMEM((_B_PER_W,), jnp.int32),
        pltpu.VMEM((_B_PER_W,), jnp.int32),
        pltpu.VMEM((2, _CHUNK, 128), jnp.float32),
        pltpu.VMEM((EMBED_DIM, _CHUNK), jnp.float32),
        pltpu.SemaphoreType.DMA,
    ],
    compiler_params=_params,
)
def _gather_rows(idx_hbm, table_hbm, out_hbm, idx_v, v_v, s_v, blocks_v,
                 ot_v, gsem):
    wid = lax.axis_index("s") * _NC + lax.axis_index("c")
    base = wid * _B_PER_W

    pltpu.sync_copy(idx_hbm.at[wid], idx_v)

    for k in range(_B_PER_W // _L):
        ivec = idx_v[pl.ds(k * _L, _L)]
        v_v[pl.ds(k * _L, _L)] = lax.shift_right_logical(ivec, 1)
        s_v[pl.ds(k * _L, _L)] = lax.shift_left(lax.bitwise_and(ivec, 1), 6)

    iota = lax.iota(jnp.int32, _L)
    i0s = [lg * 16 + iota for lg in range(_CHUNK // _L)]

    def fire(c, buf):
        return pltpu.async_copy(
            table_hbm.at[v_v.at[pl.ds(c * _CHUNK, _CHUNK)]],
            blocks_v.at[buf],
            gsem,
        )

    def extract(c, buf):
        src = blocks_v.at[buf]
        for lg in range(_CHUNK // _L):
            i0 = i0s[lg]
            s64 = s_v[pl.ds(c * _CHUNK + lg * 16, _L)]

            @plsc.parallel_loop(0, EMBED_DIM, 1, unroll=8)
            def _(d):
                x = plsc.load_gather(src, [i0, s64 + d])
                ot_v[d, pl.ds(lg * _L, _L)] = x

    cp = fire(0, 0)
    for c in range(_N_CHUNKS):
        nxt = fire(c + 1, (c + 1) % 2) if c + 1 < _N_CHUNKS else None
        cp.wait()
        extract(c, c % 2)
        pltpu.sync_copy(
            ot_v, out_hbm.at[:, pl.ds(base + c * _CHUNK, _CHUNK)]
        )
        cp = nxt


def kernel(inputs, W):
    idx = inputs.reshape(_NW, _B_PER_W)
    wp = W.reshape(_NGRP, 8, EMBED_DIM)
    table = _depad_table(wp).reshape(VOCAB // 2, 2 * EMBED_DIM)
    return _gather_rows(idx, table).T
